# baseline (speedup 1.0000x reference)
import functools

import jax
import jax.numpy as jnp
from jax import lax
from jax.experimental import pallas as pl
from jax.experimental.pallas import tpu as pltpu
from jax.experimental.pallas import tpu_sc as plsc

_NUM_BUCKETS = 32
_SCALE = 0.125
_LANES = 128
_SC_CORES = 2
_SC_SUBCORES = 16


def _make_stair_sc(seq, heads, nb, rb, width):
    rows = nb * heads * 8
    workers = _SC_CORES * _SC_SUBCORES
    assert rows % workers == 0 and width % 16 == 0
    rows_per_worker = rows // workers
    emb_n = heads * _NUM_BUCKETS
    mesh = plsc.VectorSubcoreMesh(core_axis_name="c", subcore_axis_name="s")

    @functools.partial(
        pl.kernel,
        mesh=mesh,
        out_type=jax.ShapeDtypeStruct((nb, heads, 8, width), jnp.float32),
        scratch_types=[
            pltpu.VMEM((emb_n,), jnp.float32),
            pltpu.VMEM((width,), jnp.float32),
        ],
    )
    def stair(emb_hbm, s_hbm, embbuf, rowbuf):
        wid = lax.axis_index("s") * _SC_CORES + lax.axis_index("c")
        pltpu.sync_copy(emb_hbm, embbuf)
        for i in range(emb_n // 16):
            sl = pl.ds(i * 16, 16)
            embbuf[sl] = embbuf[sl] * _SCALE
        gdn = lax.GatherDimensionNumbers(
            offset_dims=(), collapsed_slice_dims=(0,), start_index_map=(0,)
        )
        for r in range(rows_per_worker):
            row = wid * rows_per_worker + r
            t = row // (heads * 8)
            hh = (row // 8) % heads
            sr = row % 8
            krow = (nb - 1 - t) * rb - sr - (seq - 1)
            hbase = hh * _NUM_BUCKETS
            emb_lo = embbuf[pl.ds(hbase, 16)]
            emb_hi = embbuf[pl.ds(hbase + 16, 16)]

            def body(xv, _, krow=krow, emb_lo=emb_lo, emb_hi=emb_hi):
                x = xv * 16 + lax.broadcasted_iota(jnp.int32, (16,), 0)
                n = -(x + krow)
                neg = n < 0
                n = jnp.abs(n)
                large = jnp.int32(8)
                for thr in (12, 16, 23, 32, 46, 64, 91):
                    large = large + jnp.where(n >= thr, 1, 0)
                sub = jnp.where(n < 8, n, large)
                lo = lax.gather(
                    emb_lo, sub[:, None], gdn, (1,),
                    mode=lax.GatherScatterMode.PROMISE_IN_BOUNDS,
                )
                hi = lax.gather(
                    emb_hi, sub[:, None], gdn, (1,),
                    mode=lax.GatherScatterMode.PROMISE_IN_BOUNDS,
                )
                rowbuf[pl.ds(xv * 16, 16)] = jnp.where(neg, hi, lo)
                return 0

            lax.fori_loop(0, width // 16, body, 0)
            pltpu.sync_copy(rowbuf, s_hbm.at[t, hh, sr])

    return stair


def _add_kernel(seq, rb, qk_ref, s_ref, out_ref):
    for rg in range(rb // 8):
        off = (rb - 1) - 8 * rg
        bias = s_ref[0, 0, :, off : off + seq]
        out_ref[0, 0, 8 * rg : 8 * rg + 8, :] = (
            qk_ref[0, 0, 8 * rg : 8 * rg + 8, :] + bias
        )


def kernel(qk_dots, rel_emb):
    batch, heads, seq_i, seq = qk_dots.shape
    assert seq_i == seq and seq % _LANES == 0
    rb = min(seq, 1024)
    nb = seq // rb
    width = rb + seq

    emb_flat = jnp.transpose(rel_emb.astype(jnp.float32)).reshape(-1)
    stair = _make_stair_sc(seq, heads, nb, rb, width)(emb_flat)

    return pl.pallas_call(
        functools.partial(_add_kernel, seq, rb),
        grid=(batch, heads, nb),
        in_specs=[
            pl.BlockSpec((1, 1, rb, seq), lambda b, h, t: (b, h, t, 0)),
            pl.BlockSpec((1, 1, 8, width), lambda b, h, t: (t, h, 0, 0)),
        ],
        out_specs=pl.BlockSpec((1, 1, rb, seq), lambda b, h, t: (b, h, t, 0)),
        out_shape=jax.ShapeDtypeStruct((batch, heads, seq, seq), jnp.float32),
        compiler_params=pltpu.CompilerParams(
            dimension_semantics=("parallel", "parallel", "arbitrary")
        ),
    )(qk_dots, stair)
